# baseline (speedup 1.0000x reference)
import functools

import jax
import jax.numpy as jnp
from jax import lax
from jax.experimental import pallas as pl
from jax.experimental.pallas import tpu as pltpu
from jax.experimental.pallas import tpu_sc as plsc

N = 100000
D = 128

NC = 2
NS = 16
NW = NC * NS

PER_W = 3120
EXTRA = 16
NEXTRA = 10
CHUNK = 312
NCHUNK = PER_W // CHUNK
NBUF = 3


def _emb_body(idx_hbm, tbl_hbm, out_hbm,
              tbl_sh, idx_all, r0, r1, r2, tr,
              isem, g0, g1, g2, o0, o1, o2):
  row_bufs = (r0, r1, r2)
  gsem = (g0, g1, g2)
  osem = (o0, o1, o2)

  sid = lax.axis_index("s")
  wid = sid * NC + lax.axis_index("c")
  base = wid * PER_W + jnp.minimum(wid, NEXTRA) * EXTRA

  idx_cp = pltpu.async_copy(idx_hbm.at[pl.ds(base, PER_W + EXTRA)], idx_all,
                            isem)

  @pl.when(sid == 0)
  def _stage():
    pltpu.sync_copy(tbl_hbm, tbl_sh.at[pl.ds(1, 93)])

  plsc.subcore_barrier()
  idx_cp.wait()

  gathers = [None] * NBUF

  def fire(c, b):
    gathers[b] = pltpu.async_copy(
        tbl_sh.at[idx_all.at[pl.ds(c * CHUNK, CHUNK)]], row_bufs[b], gsem[b])

  def out_fire(c, b):
    pltpu.async_copy(row_bufs[b],
                     out_hbm.at[pl.ds(base + c * CHUNK, CHUNK)], osem[b])

  def out_wait(b):
    pltpu.make_async_copy(row_bufs[b],
                          out_hbm.at[pl.ds(base, CHUNK)], osem[b]).wait()

  def grp(i, carry):
    c0 = i * NBUF
    for b in range(NBUF):
      @pl.when(i > 0)
      def _reuse(b=b):
        out_wait(b)
      fire(c0 + b, b)
    for b in range(NBUF):
      gathers[b].wait()
      out_fire(c0 + b, b)
    return carry

  lax.fori_loop(0, NCHUNK // NBUF, grp, 0)
  out_wait(0)
  fire(NCHUNK - 1, 0)
  gathers[0].wait()
  out_fire(NCHUNK - 1, 0)

  @pl.when(wid < NEXTRA)
  def _tail():
    pltpu.async_copy(tbl_sh.at[idx_all.at[pl.ds(PER_W, EXTRA)]], tr,
                     g0).wait()
    pltpu.sync_copy(tr, out_hbm.at[pl.ds(base + PER_W, EXTRA)])

  for b in range(NBUF):
    out_wait(b)


@jax.jit
def kernel(x, table):
  idx = jnp.reshape(x, (N,)).astype(jnp.int32)

  mesh = plsc.VectorSubcoreMesh(core_axis_name="c", subcore_axis_name="s")
  run = pl.kernel(
      _emb_body,
      mesh=mesh,
      out_type=jax.ShapeDtypeStruct((N, D), jnp.float32),
      scratch_types=[
          pltpu.VMEM_SHARED((94, D), jnp.float32),
          pltpu.VMEM((PER_W + EXTRA,), jnp.int32),
          pltpu.VMEM((CHUNK, D), jnp.float32),
          pltpu.VMEM((CHUNK, D), jnp.float32),
          pltpu.VMEM((CHUNK, D), jnp.float32),
          pltpu.VMEM((EXTRA, D), jnp.float32),
          pltpu.SemaphoreType.DMA,
          pltpu.SemaphoreType.DMA,
          pltpu.SemaphoreType.DMA,
          pltpu.SemaphoreType.DMA,
          pltpu.SemaphoreType.DMA,
          pltpu.SemaphoreType.DMA,
          pltpu.SemaphoreType.DMA,
      ],
  )
  return run(idx, table)


# speedup vs baseline: 1.0778x; 1.0778x over previous
import jax
import jax.numpy as jnp
from jax import lax
from jax.experimental import pallas as pl
from jax.experimental.pallas import tpu as pltpu
from jax.experimental.pallas import tpu_sc as plsc

N = 100000
D = 128

NC = 2
NS = 16
NW = NC * NS

PER_W = 3120
EXTRA = 16
NEXTRA = 10
CHUNK = 312
NCHUNK = PER_W // CHUNK
NBUF = 3


def _emb_body(idx_hbm, tbl_hbm, out_hbm,
              tbl_sh, idx_all, r0, r1, r2, tr,
              isem, g0, g1, g2, o0, o1, o2):
  row_bufs = (r0, r1, r2)
  gsem = (g0, g1, g2)
  osem = (o0, o1, o2)

  sid = lax.axis_index("s")
  wid = sid * NC + lax.axis_index("c")
  base = wid * PER_W + jnp.minimum(wid, NEXTRA) * EXTRA

  idx_cp = pltpu.async_copy(idx_hbm.at[pl.ds(base, PER_W)],
                            idx_all.at[pl.ds(0, PER_W)], isem)

  @pl.when(wid < NEXTRA)
  def _tail_idx():
    pltpu.async_copy(idx_hbm.at[pl.ds(base + PER_W, EXTRA)],
                     idx_all.at[pl.ds(PER_W, EXTRA)], isem)

  @pl.when(sid == 0)
  def _stage():
    pltpu.sync_copy(tbl_hbm, tbl_sh.at[pl.ds(1, 93)])

  plsc.subcore_barrier()
  idx_cp.wait()

  @pl.when(wid < NEXTRA)
  def _tail_idx_wait():
    pltpu.make_async_copy(idx_hbm.at[pl.ds(base + PER_W, EXTRA)],
                          idx_all.at[pl.ds(PER_W, EXTRA)], isem).wait()

  gathers = [None] * NBUF
  outs = [None] * NBUF

  def issue(g, b):
    gathers[b] = pltpu.async_copy(
        tbl_sh.at[idx_all.at[pl.ds(g * CHUNK, CHUNK)]], row_bufs[b], gsem[b])

  for g in range(NBUF - 1):
    issue(g, g)

  for g in range(NCHUNK):
    b = g % NBUF
    la = g + NBUF - 1
    if la < NCHUNK:
      lb = la % NBUF
      if la >= NBUF:
        outs[lb].wait()
      issue(la, lb)
    gathers[b].wait()
    outs[b] = pltpu.async_copy(row_bufs[b],
                               out_hbm.at[pl.ds(base + g * CHUNK, CHUNK)],
                               osem[b])
  for g in range(max(0, NCHUNK - NBUF), NCHUNK):
    outs[g % NBUF].wait()

  @pl.when(wid < NEXTRA)
  def _tail():
    pltpu.async_copy(tbl_sh.at[idx_all.at[pl.ds(PER_W, EXTRA)]], tr,
                     g0).wait()
    pltpu.sync_copy(tr, out_hbm.at[pl.ds(base + PER_W, EXTRA)])


@jax.jit
def kernel(x, table):
  idx = jnp.reshape(x, (N,)).astype(jnp.int32)

  mesh = plsc.VectorSubcoreMesh(core_axis_name="c", subcore_axis_name="s")
  run = pl.kernel(
      _emb_body,
      mesh=mesh,
      out_type=jax.ShapeDtypeStruct((N, D), jnp.float32),
      scratch_types=[
          pltpu.VMEM_SHARED((94, D), jnp.float32),
          pltpu.VMEM((PER_W + EXTRA,), jnp.int32),
          pltpu.VMEM((CHUNK, D), jnp.float32),
          pltpu.VMEM((CHUNK, D), jnp.float32),
          pltpu.VMEM((CHUNK, D), jnp.float32),
          pltpu.VMEM((EXTRA, D), jnp.float32),
          pltpu.SemaphoreType.DMA,
          pltpu.SemaphoreType.DMA,
          pltpu.SemaphoreType.DMA,
          pltpu.SemaphoreType.DMA,
          pltpu.SemaphoreType.DMA,
          pltpu.SemaphoreType.DMA,
          pltpu.SemaphoreType.DMA,
      ],
  )
  return run(idx, table)
